# baseline (speedup 1.0000x reference)
import functools

import jax
import jax.numpy as jnp
from jax import lax
from jax.experimental import pallas as pl
from jax.experimental.pallas import tpu as pltpu
from jax.experimental.pallas import tpu_sc as plsc

_H = 200
_BATCH = 4096
_D = 64
_BB = 128
_NB = 4


@functools.lru_cache(maxsize=None)
def _make():
    info = plsc.get_sparse_core_info()
    nw = info.num_cores * info.num_subcores
    assert nw * _BB == _BATCH

    mesh = plsc.VectorSubcoreMesh(core_axis_name="c", subcore_axis_name="s")

    @functools.partial(
        pl.kernel,
        mesh=mesh,
        out_type=jax.ShapeDtypeStruct((_H, 8, 32, 8, 128), jnp.float32),
        scratch_types=(
            [pltpu.VMEM((_H, _BB), jnp.int32)]
            + [pltpu.VMEM((_BB,), jnp.int32) for _ in range(_NB)]
            + [pltpu.VMEM((_BB, _D), jnp.float32) for _ in range(_NB)]
            + [pltpu.VMEM((_D, _BB), jnp.float32) for _ in range(_NB)]
            + [pltpu.SemaphoreType.DMA for _ in range(2 * _NB)]
        ),
        compiler_params=pltpu.CompilerParams(
            use_tc_tiling_on_sc=False, needs_layout_passes=False,
            disable_bounds_checks=True),
    )
    def k(xst_hbm, table_hbm, out_hbm, idxt, *bufs):
        idx2 = bufs[0:_NB]
        grows = bufs[_NB:2 * _NB]
        ot = bufs[2 * _NB:3 * _NB]
        sg = bufs[3 * _NB:4 * _NB]
        sw = bufs[4 * _NB:5 * _NB]
        wid = lax.axis_index("s") * info.num_cores + lax.axis_index("c")
        col = wid * _BB

        iota = lax.iota(jnp.int32, 16)
        lanes = [iota + 16 * j for j in range(8)]

        pltpu.sync_copy(xst_hbm.at[:, pl.ds(col, _BB)], idxt)

        def prep(h, b):
            hv = jnp.full((16,), h, jnp.int32)
            for j in range(8):
                v = plsc.load_gather(idxt, [hv, lanes[j]])
                idx2[b][pl.ds(16 * j, 16)] = v

        def g_start(b):
            pltpu.async_copy(table_hbm.at[idx2[b]], grows[b], sg[b])

        def g_wait(b):
            pltpu.make_async_copy(table_hbm.at[idx2[b]], grows[b], sg[b]).wait()

        def w_start(h, b):
            for fr in range(8):
                pltpu.async_copy(
                    ot[b].at[pl.ds(8 * fr, 8), :], out_hbm.at[h, fr, wid],
                    sw[b])

        def w_wait(b):
            for fr in range(8):
                pltpu.make_async_copy(
                    ot[b].at[pl.ds(8 * fr, 8), :], out_hbm.at[0, fr, wid],
                    sw[b]).wait()

        def extract(b):
            def blk(t, carry):
                rowv = iota + (t % 8) * 16
                f0 = (t // 8) * 16
                for l in range(16):
                    fv = jnp.full((16,), f0 + l, jnp.int32)
                    val = plsc.load_gather(grows[b], [rowv, fv])
                    plsc.store_scatter(ot[b], [fv, rowv], val)
                return carry

            lax.fori_loop(0, 32, blk, 0)

        for h in range(_NB):
            prep(h, h)
            g_start(h)
        for h in range(_NB):
            g_wait(h)
            extract(h)
            w_start(h, h)
            prep(h + _NB, h)
            g_start(h)

        def body(g, carry):
            for b in range(_NB):
                h = _NB * g + b
                g_wait(b)
                w_wait(b)
                extract(b)
                w_start(h, b)
                prep(h + _NB, b)
                g_start(b)
            return carry

        lax.fori_loop(1, (_H - _NB) // _NB, body, 0)

        for hh in range(_H - _NB, _H):
            b = hh % _NB
            g_wait(b)
            w_wait(b)
            extract(b)
            w_start(hh, b)
        for b in range(_NB):
            w_wait(b)

    return k


def kernel(xs, table):
    out5 = _make()(xs.T, table)
    return out5.transpose(2, 4, 0, 1, 3).reshape(_BATCH, _H, _D)


# speedup vs baseline: 1.0050x; 1.0050x over previous
import functools

import jax
import jax.numpy as jnp
from jax import lax
from jax.experimental import pallas as pl
from jax.experimental.pallas import tpu as pltpu
from jax.experimental.pallas import tpu_sc as plsc

_H = 200
_BATCH = 4096
_D = 64
_BB = 128
_NB = 4


@functools.lru_cache(maxsize=None)
def _make():
    info = plsc.get_sparse_core_info()
    nw = info.num_cores * info.num_subcores
    assert nw * _BB == _BATCH

    mesh = plsc.VectorSubcoreMesh(core_axis_name="c", subcore_axis_name="s")

    @functools.partial(
        pl.kernel,
        mesh=mesh,
        out_type=jax.ShapeDtypeStruct((_H, 8, 32, 8, 128), jnp.float32),
        scratch_types=(
            [pltpu.VMEM((_H, _BB), jnp.int32)]
            + [pltpu.VMEM((_BB, _D), jnp.float32) for _ in range(_NB)]
            + [pltpu.VMEM((_D, _BB), jnp.float32) for _ in range(_NB)]
            + [pltpu.SemaphoreType.DMA for _ in range(2 * _NB)]
        ),
        compiler_params=pltpu.CompilerParams(
            use_tc_tiling_on_sc=False, needs_layout_passes=False,
            disable_bounds_checks=True),
    )
    def k(xst_hbm, table_hbm, out_hbm, idxt, *bufs):
        grows = bufs[0:_NB]
        ot = bufs[_NB:2 * _NB]
        sg = bufs[2 * _NB:3 * _NB]
        sw = bufs[3 * _NB:4 * _NB]
        wid = lax.axis_index("s") * info.num_cores + lax.axis_index("c")
        col = wid * _BB

        iota = lax.iota(jnp.int32, 16)
        lanes = [iota + 16 * j for j in range(8)]

        pltpu.sync_copy(xst_hbm.at[:, pl.ds(col, _BB)], idxt)

        def g_start(h, b):
            pltpu.async_copy(table_hbm.at[idxt.at[h]], grows[b], sg[b])

        def g_wait(b):
            pltpu.make_async_copy(
                table_hbm.at[idxt.at[0]], grows[b], sg[b]).wait()

        def w_start(h, b):
            for fr in range(8):
                pltpu.async_copy(
                    ot[b].at[pl.ds(8 * fr, 8), :], out_hbm.at[h, fr, wid],
                    sw[b])

        def w_wait(b):
            for fr in range(8):
                pltpu.make_async_copy(
                    ot[b].at[pl.ds(8 * fr, 8), :], out_hbm.at[0, fr, wid],
                    sw[b]).wait()

        zero16 = jnp.zeros((16,), jnp.int32)

        def extract(b):
            def blk(t, carry):
                rowv = iota + (t % 8) * 16
                row64 = rowv * 64
                f0 = (t // 8) * 16
                for l in range(16):
                    fl = f0 + l
                    val = plsc.load_gather(grows[b], [zero16, row64 + fl])
                    plsc.store_scatter(ot[b], [zero16, rowv + fl * 128], val)
                return carry

            lax.fori_loop(0, 32, blk, 0)

        for h in range(_NB):
            g_start(h, h)
        for h in range(_NB):
            g_wait(h)
            extract(h)
            w_start(h, h)
            g_start(h + _NB, h)

        def body(g, carry):
            for b in range(_NB):
                h = _NB * g + b
                g_wait(b)
                w_wait(b)
                extract(b)
                w_start(h, b)
                g_start(h + _NB, b)
            return carry

        lax.fori_loop(1, (_H - _NB) // _NB, body, 0)

        for hh in range(_H - _NB, _H):
            b = hh % _NB
            g_wait(b)
            w_wait(b)
            extract(b)
            w_start(hh, b)
        for b in range(_NB):
            w_wait(b)

    return k


def kernel(xs, table):
    out5 = _make()(xs.T, table)
    return out5.transpose(2, 4, 0, 1, 3).reshape(_BATCH, _H, _D)


# speedup vs baseline: 1.0107x; 1.0057x over previous
import functools

import jax
import jax.numpy as jnp
from jax import lax
from jax.experimental import pallas as pl
from jax.experimental.pallas import tpu as pltpu
from jax.experimental.pallas import tpu_sc as plsc

_H = 200
_BATCH = 4096
_D = 64
_BB = 128
_NH = 2
_STEPS = _H // _NH


@functools.lru_cache(maxsize=None)
def _make():
    info = plsc.get_sparse_core_info()
    nw = info.num_cores * info.num_subcores
    assert nw * _BB == _BATCH

    mesh = plsc.VectorSubcoreMesh(core_axis_name="c", subcore_axis_name="s")

    @functools.partial(
        pl.kernel,
        mesh=mesh,
        out_type=jax.ShapeDtypeStruct((_H, 8, 32, 8, 128), jnp.float32),
        scratch_types=(
            [pltpu.VMEM((_H, _BB), jnp.int32)]
            + [pltpu.VMEM((_NH, _BB, _D), jnp.float32) for _ in range(2)]
            + [pltpu.VMEM((_NH, 8, 8, 128), jnp.float32) for _ in range(2)]
            + [pltpu.SemaphoreType.DMA for _ in range(4)]
        ),
        compiler_params=pltpu.CompilerParams(
            use_tc_tiling_on_sc=False, needs_layout_passes=False,
            disable_bounds_checks=True),
    )
    def k(xst_hbm, table_hbm, out_hbm, idxt, g0, g1, o0, o1, sg0, sg1, sw0, sw1):
        grows = (g0, g1)
        ot = (o0, o1)
        sg = (sg0, sg1)
        sw = (sw0, sw1)
        wid = lax.axis_index("s") * info.num_cores + lax.axis_index("c")
        col = wid * _BB

        iota = lax.iota(jnp.int32, 16)
        zero16 = jnp.zeros((16,), jnp.int32)

        pltpu.sync_copy(xst_hbm.at[:, pl.ds(col, _BB)], idxt)

        def g_start(s, b):
            for hh in range(_NH):
                pltpu.async_copy(
                    table_hbm.at[idxt.at[_NH * s + hh]], grows[b].at[hh],
                    sg[b])

        def g_wait(b):
            for hh in range(_NH):
                pltpu.make_async_copy(
                    table_hbm.at[idxt.at[0]], grows[b].at[hh], sg[b]).wait()

        def w_start(s, b):
            pltpu.async_copy(
                ot[b], out_hbm.at[pl.ds(_NH * s, _NH), :, wid, :, :], sw[b])

        def w_wait(b):
            pltpu.make_async_copy(
                ot[b], out_hbm.at[pl.ds(0, _NH), :, wid, :, :], sw[b]).wait()

        def extract(b):
            def blk(t, carry):
                hh = t >> 5
                tt = t & 31
                rowv = iota + (tt % 8) * 16
                src0 = rowv * 64 + hh * (_BB * _D)
                f0 = (tt // 8) * 16
                for l in range(16):
                    fl = f0 + l
                    off = hh * 8192 + (fl >> 3) * 1024 + (fl & 7) * 128
                    val = plsc.load_gather(
                        grows[b], [zero16, zero16, src0 + fl])
                    plsc.store_scatter(
                        ot[b], [zero16, zero16, zero16, rowv + off], val)
                return carry

            lax.fori_loop(0, 32 * _NH, blk, 0)

        for s in range(2):
            g_start(s, s)
        for s in range(2):
            g_wait(s)
            extract(s)
            w_start(s, s)
            g_start(s + 2, s)

        def body(g, carry):
            for b in range(2):
                s = 2 * g + b
                g_wait(b)
                w_wait(b)
                extract(b)
                w_start(s, b)
                g_start(s + 2, b)
            return carry

        lax.fori_loop(1, _STEPS // 2 - 1, body, 0)

        for s in (_STEPS - 2, _STEPS - 1):
            b = s % 2
            g_wait(b)
            w_wait(b)
            extract(b)
            w_start(s, b)
        w_wait(0)
        w_wait(1)

    return k


def kernel(xs, table):
    out5 = _make()(xs.T, table)
    return out5.transpose(2, 4, 0, 1, 3).reshape(_BATCH, _H, _D)
